# baseline (speedup 1.0000x reference)
import functools

import jax
import jax.numpy as jnp
from jax import lax
from jax.experimental import pallas as pl
from jax.experimental.pallas import tpu as pltpu
from jax.experimental.pallas import tpu_sc as plsc

_INPUT_SIZE = 768
_NUM_EXPERTS = 64
_BT = 4096
_NW = 32
_LANES = 16


def _mm_t_body(x_ref, wt_ref, b_ref, out_ref):
    s = jnp.dot(x_ref[...], wt_ref[...],
                preferred_element_type=jnp.float32)
    out_ref[...] = s.T + b_ref[...]


def _tc_scores_t(x, wt, b2, n_tokens):
    grid = (n_tokens // _BT,)
    return pl.pallas_call(
        _mm_t_body,
        grid=grid,
        in_specs=[
            pl.BlockSpec((_BT, _INPUT_SIZE), lambda i: (i, 0)),
            pl.BlockSpec((_INPUT_SIZE, _NUM_EXPERTS), lambda i: (0, 0)),
            pl.BlockSpec((_NUM_EXPERTS, 1), lambda i: (0, 0)),
        ],
        out_specs=pl.BlockSpec((_NUM_EXPERTS, _BT), lambda i: (0, i)),
        out_shape=jax.ShapeDtypeStruct((_NUM_EXPERTS, n_tokens), jnp.float32),
        compiler_params=pltpu.CompilerParams(
            dimension_semantics=("arbitrary",),
        ),
    )(x, wt, b2)


def _sc_route(scores_t, n_tokens):
    chunk = n_tokens // _NW
    n_groups = chunk // _LANES
    mesh = plsc.VectorSubcoreMesh(core_axis_name="c", subcore_axis_name="s")

    @functools.partial(
        pl.kernel,
        mesh=mesh,
        out_type=[
            jax.ShapeDtypeStruct((2, n_tokens), jnp.float32),
            jax.ShapeDtypeStruct((2, n_tokens), jnp.int32),
        ],
        scratch_types=[
            pltpu.VMEM((_NUM_EXPERTS, chunk), jnp.float32),
            pltpu.VMEM((chunk,), jnp.float32),
            pltpu.VMEM((chunk,), jnp.float32),
            pltpu.VMEM((chunk,), jnp.int32),
            pltpu.VMEM((chunk,), jnp.int32),
        ],
    )
    def route(st_hbm, outp_hbm, outi_hbm, st_v, p1_v, p2_v, i1_v, i2_v):
        wid = lax.axis_index("s") * 2 + lax.axis_index("c")
        base = wid * chunk
        pltpu.sync_copy(st_hbm.at[:, pl.ds(base, chunk)], st_v)

        lane = jnp.arange(_LANES, dtype=jnp.int32)
        zeros = jnp.zeros((_LANES,), jnp.int32)
        neg_inf = jnp.full((_LANES,), -jnp.inf, jnp.float32)

        def group_body(g, carry):
            t0 = g * _LANES
            m1 = neg_inf
            m2 = neg_inf
            i1 = zeros
            i2 = zeros
            for e in range(_NUM_EXPERTS):
                v = st_v[e, pl.ds(t0, _LANES)]
                e_vec = jnp.full((_LANES,), e, jnp.int32)
                gt1 = v > m1
                gt2 = v > m2
                i2 = jnp.where(gt1, i1, jnp.where(gt2, e_vec, i2))
                m2 = jnp.where(gt1, m1, jnp.where(gt2, v, m2))
                i1 = jnp.where(gt1, e_vec, i1)
                m1 = jnp.where(gt1, v, m1)
            ex = jnp.exp(m2 - m1)
            denom = 1.0 + ex
            p1_v[pl.ds(t0, _LANES)] = 1.0 / denom
            p2_v[pl.ds(t0, _LANES)] = ex / denom
            i1_v[pl.ds(t0, _LANES)] = i1
            i2_v[pl.ds(t0, _LANES)] = i2
            return carry

        lax.fori_loop(0, n_groups, group_body, 0)
        pltpu.sync_copy(p1_v, outp_hbm.at[0, pl.ds(base, chunk)])
        pltpu.sync_copy(p2_v, outp_hbm.at[1, pl.ds(base, chunk)])
        pltpu.sync_copy(i1_v, outi_hbm.at[0, pl.ds(base, chunk)])
        pltpu.sync_copy(i2_v, outi_hbm.at[1, pl.ds(base, chunk)])

    p_t, i_t = route(scores_t)
    return p_t.T, i_t.T


def kernel(x, W, b):
    n_tokens = x.shape[0]
    wt = W.T
    b2 = b.reshape(_NUM_EXPERTS, 1)
    scores_t = _tc_scores_t(x, wt, b2, n_tokens)
    return _sc_route(scores_t, n_tokens)
